# Initial kernel scaffold; baseline (speedup 1.0000x reference)
import jax
import jax.numpy as jnp
from jax.experimental import pallas as pl


def kernel(x, edge_index, W0, b0, W1, b1, W2, b2):
    raise NotImplementedError("write your pallas kernel here")


# speedup vs baseline: 11.8155x; 11.8155x over previous
import jax
import jax.numpy as jnp
from jax import lax
from jax.experimental import pallas as pl
from jax.experimental.pallas import tpu as pltpu
from jax.experimental.pallas import tpu_sc as plsc

N = 10000
NPAD = 10240
E = 160000
NFEAT = 256
NHID = 512
NCLASS = 256

NC, NS, LANES = 2, 16, 16
NW = NC * NS
EPW = E // NW
EPT = E // NS
CH = 80
NCHUNK = EPT // CH
NBUF = 5
RPT = NPAD // NS
RCHUNK = 160
FBW = 64
RT = 512


def _hist_body(row_ref, out_ref, idx_v, hist_v):
    c = lax.axis_index("c")
    s = lax.axis_index("s")
    w = s * NC + c
    pltpu.sync_copy(row_ref.at[w], idx_v)
    zero16 = jnp.zeros((LANES,), jnp.float32)

    def zbody(i, _):
        hist_v[pl.ds(i * LANES, LANES)] = zero16
        return ()

    lax.fori_loop(0, NPAD // LANES, zbody, ())
    ones = jnp.ones((LANES,), jnp.float32)

    def body(i, _):
        idx = idx_v[pl.ds(i * LANES, LANES)]
        plsc.addupdate_scatter(hist_v, [idx], ones)
        return ()

    nfull = EPW // LANES
    lax.fori_loop(0, nfull, body, ())
    tail = idx_v[pl.ds(EPW - LANES, LANES)]
    m = lax.iota(jnp.int32, LANES) >= (LANES - (EPW - nfull * LANES))
    plsc.addupdate_scatter(hist_v, [tail], ones, mask=m)
    pltpu.sync_copy(hist_v, out_ref.at[w])


_hist = pl.kernel(
    _hist_body,
    out_type=jax.ShapeDtypeStruct((NW, NPAD), jnp.float32),
    mesh=plsc.VectorSubcoreMesh(
        core_axis_name="c", subcore_axis_name="s", num_cores=NC, num_subcores=NS
    ),
    compiler_params=pltpu.CompilerParams(needs_layout_passes=False),
    scratch_types=[
        pltpu.VMEM((EPW,), jnp.int32),
        pltpu.VMEM((NPAD,), jnp.float32),
    ],
)


def _spmm_body(g_ref, col_ref, row_ref, out_ref,
               cbuf, rbuf, dbuf, obuf, zbuf, acc, *gsems):
    c = lax.axis_index("c")
    s = lax.axis_index("s")
    zero16 = jnp.zeros((LANES,), jnp.float32)
    nz = FBW // LANES

    def zb(i, _):
        zbuf[i // nz, pl.ds((i % nz) * LANES, LANES)] = zero16
        return ()

    lax.fori_loop(0, RCHUNK * nz, zb, ())
    pltpu.sync_copy(row_ref.at[s], rbuf)
    pltpu.sync_copy(col_ref.at[c, s], cbuf)
    for k in range(RPT // RCHUNK):
        pltpu.sync_copy(zbuf, acc.at[pl.ds(s * RPT + k * RCHUNK, RCHUNK)])
    plsc.subcore_barrier()

    for b in range(NBUF):
        pltpu.async_copy(g_ref.at[cbuf.at[b]], dbuf.at[b], gsems[b])

    def grp(gi, _):
        for b in range(NBUF):
            i = gi * NBUF + b
            pltpu.make_async_copy(
                g_ref.at[cbuf.at[i]], dbuf.at[b], gsems[b]
            ).wait()
            pltpu.sync_copy(dbuf.at[b], acc.at[rbuf.at[i]], add=True)

            @pl.when(i + NBUF < NCHUNK)
            def _():
                pltpu.async_copy(
                    g_ref.at[cbuf.at[i + NBUF]], dbuf.at[b], gsems[b]
                )
        return ()

    lax.fori_loop(0, NCHUNK // NBUF, grp, ())
    plsc.subcore_barrier()
    for k in range(RPT // RCHUNK):
        pltpu.sync_copy(acc.at[pl.ds(s * RPT + k * RCHUNK, RCHUNK)], obuf)
        pltpu.sync_copy(
            obuf, out_ref.at[c, pl.ds(s * RPT + k * RCHUNK, RCHUNK)]
        )


_spmm = pl.kernel(
    _spmm_body,
    out_type=jax.ShapeDtypeStruct((NC, NPAD, FBW), jnp.float32),
    mesh=plsc.VectorSubcoreMesh(
        core_axis_name="c", subcore_axis_name="s", num_cores=NC, num_subcores=NS
    ),
    compiler_params=pltpu.CompilerParams(
        needs_layout_passes=False, use_tc_tiling_on_sc=False
    ),
    scratch_types=[
        pltpu.VMEM((NCHUNK, CH), jnp.int32),
        pltpu.VMEM((NCHUNK, CH), jnp.int32),
        pltpu.VMEM((NBUF, CH, FBW), jnp.float32),
        pltpu.VMEM((RCHUNK, FBW), jnp.float32),
        pltpu.VMEM((RCHUNK, FBW), jnp.float32),
        pltpu.VMEM_SHARED((NPAD, FBW), jnp.float32),
    ] + [pltpu.SemaphoreType.DMA] * NBUF,
)


def _l0_body(pt_ref, x_ref, w_ref, b_ref, g_ref, s_ref):
    deg = jnp.sum(pt_ref[...], axis=1, keepdims=True) + 1.0
    scol = lax.rsqrt(deg)
    h = jnp.dot(x_ref[...], w_ref[...], preferred_element_type=jnp.float32)
    g_ref[0] = scol * (h + b_ref[...])
    s_ref[...] = jnp.broadcast_to(scol, (RT, 128))


def _mid_body(t0, t1, t2, t3, g_ref, s_ref, w_ref, b_ref, o_ref):
    sb = s_ref[...]
    acc = jnp.zeros((RT, 128), jnp.float32)
    for fb, t_ref in enumerate((t0, t1, t2, t3)):
        t = jnp.concatenate([t_ref[0], t_ref[1]], axis=1)
        z = jnp.maximum(sb * (t + g_ref[fb]), 0.0)
        acc = acc + jnp.dot(
            z, w_ref[...][fb * 128:(fb + 1) * 128, :],
            preferred_element_type=jnp.float32,
        )
    o_ref[0] = sb * (acc + b_ref[...])


def _fin_body(t0, t1, g_ref, s_ref, o_ref):
    for k, t_ref in enumerate((t0, t1)):
        t = jnp.concatenate([t_ref[0], t_ref[1]], axis=1)
        o_ref[:, pl.ds(k * 128, 128)] = s_ref[...] * (t + g_ref[k])


_NT = NPAD // RT
_TSPEC = pl.BlockSpec((NC, RT, FBW), lambda k, i: (0, i, 0))

_l0 = pl.pallas_call(
    _l0_body,
    grid=(NHID // 128, _NT),
    in_specs=[
        pl.BlockSpec((RT, NW), lambda k, i: (i, 0)),
        pl.BlockSpec((RT, NFEAT), lambda k, i: (i, 0)),
        pl.BlockSpec((NFEAT, 128), lambda k, i: (0, k)),
        pl.BlockSpec((1, 128), lambda k, i: (0, k)),
    ],
    out_specs=[
        pl.BlockSpec((1, RT, 128), lambda k, i: (k, i, 0)),
        pl.BlockSpec((RT, 128), lambda k, i: (i, 0)),
    ],
    out_shape=[
        jax.ShapeDtypeStruct((NHID // 128, NPAD, 128), jnp.float32),
        jax.ShapeDtypeStruct((NPAD, 128), jnp.float32),
    ],
)


def _make_mid(kout):
    return pl.pallas_call(
        _mid_body,
        grid=(kout, _NT),
        in_specs=[
            _TSPEC, _TSPEC, _TSPEC, _TSPEC,
            pl.BlockSpec((NHID // 128, RT, 128), lambda k, i: (0, i, 0)),
            pl.BlockSpec((RT, 128), lambda k, i: (i, 0)),
            pl.BlockSpec((NHID, 128), lambda k, i: (0, k)),
            pl.BlockSpec((1, 128), lambda k, i: (0, k)),
        ],
        out_specs=pl.BlockSpec((1, RT, 128), lambda k, i: (k, i, 0)),
        out_shape=jax.ShapeDtypeStruct((NHID // 128, NPAD, 128), jnp.float32),
    )


_mid1 = _make_mid(NHID // 128)
_mid2 = _make_mid(NCLASS // 128)

_fin = pl.pallas_call(
    _fin_body,
    grid=(_NT,),
    in_specs=[
        pl.BlockSpec((NC, RT, FBW), lambda i: (0, i, 0)),
        pl.BlockSpec((NC, RT, FBW), lambda i: (0, i, 0)),
        pl.BlockSpec((NCLASS // 128, RT, 128), lambda i: (0, i, 0)),
        pl.BlockSpec((RT, 128), lambda i: (i, 0)),
    ],
    out_specs=pl.BlockSpec((RT, NCLASS), lambda i: (i, 0)),
    out_shape=jax.ShapeDtypeStruct((NPAD, NCLASS), jnp.float32),
)


def kernel(x, edge_index, W0, b0, W1, b1, W2, b2):
    row = edge_index[0].astype(jnp.int32)
    col = edge_index[1].astype(jnp.int32)
    row_h = row.reshape(NW, EPW)
    row_rs = row.reshape(NS, NCHUNK, CH)
    col2 = 2 * col
    cidx = []
    for k in range(4):
        per_core = jnp.stack(
            [col2 + (2 * k * NPAD + c) for c in range(NC)]
        )
        cidx.append(per_core.reshape(NC, NS, NCHUNK, CH))
    x_pad = jnp.pad(x, ((0, NPAD - N), (0, 0)))

    partials_t = _hist(row_h).T
    g1, sblk = _l0(partials_t, x_pad, W0, b0.reshape(1, NHID))
    g1f = g1.reshape(8 * NPAD, FBW)
    t1 = [_spmm(g1f, cidx[k], row_rs) for k in range(4)]
    g2 = _mid1(*t1, g1, sblk, W1, b1.reshape(1, NHID))
    g2f = g2.reshape(8 * NPAD, FBW)
    t2 = [_spmm(g2f, cidx[k], row_rs) for k in range(4)]
    g3 = _mid2(*t2, g2, sblk, W2, b2.reshape(1, NCLASS))
    t3 = [_spmm(g3.reshape(8 * NPAD, FBW), cidx[k], row_rs) for k in range(2)]
    out = _fin(*t3, g3, sblk)
    return out[:N]


# speedup vs baseline: 13.1011x; 1.1088x over previous
import jax
import jax.numpy as jnp
from jax import lax
from jax.experimental import pallas as pl
from jax.experimental.pallas import tpu as pltpu
from jax.experimental.pallas import tpu_sc as plsc

N = 10000
NPAD = 10240
E = 160000
NFEAT = 256
NHID = 512
NCLASS = 256

NC, NS, LANES = 2, 16, 16
NW = NC * NS
EPW = E // NW
EPT = E // NS
CH = 80
NCHUNK = EPT // CH
NBUF = 5
RPT = NPAD // NS
RCHUNK = 160
FBW = 64
RT = 512

_MESH = plsc.VectorSubcoreMesh(
    core_axis_name="c", subcore_axis_name="s", num_cores=NC, num_subcores=NS
)


def _hist_loop(idx_v, hist_v, val_fn):
    zero16 = jnp.zeros((LANES,), jnp.float32)

    def zbody(i, _):
        hist_v[pl.ds(i * LANES, LANES)] = zero16
        return ()

    lax.fori_loop(0, NPAD // LANES, zbody, ())

    def body(i, _):
        idx = idx_v[pl.ds(i * LANES, LANES)]
        plsc.addupdate_scatter(hist_v, [idx], val_fn(i * LANES))
        return ()

    nfull = EPW // LANES
    lax.fori_loop(0, nfull, body, ())
    tail = idx_v[pl.ds(EPW - LANES, LANES)]
    m = lax.iota(jnp.int32, LANES) >= (LANES - (EPW - nfull * LANES))
    plsc.addupdate_scatter(hist_v, [tail], val_fn(EPW - LANES), mask=m)


def _hist_body(row_ref, out_ref, idx_v, hist_v):
    w = lax.axis_index("s") * NC + lax.axis_index("c")
    pltpu.sync_copy(row_ref.at[w], idx_v)
    ones = jnp.ones((LANES,), jnp.float32)
    _hist_loop(idx_v, hist_v, lambda base: ones)
    pltpu.sync_copy(hist_v, out_ref.at[w])


_hist = pl.kernel(
    _hist_body,
    out_type=jax.ShapeDtypeStruct((NW, NPAD), jnp.float32),
    mesh=_MESH,
    compiler_params=pltpu.CompilerParams(needs_layout_passes=False),
    scratch_types=[
        pltpu.VMEM((EPW,), jnp.int32),
        pltpu.VMEM((NPAD,), jnp.float32),
    ],
)


def _rhist_body(row_ref, col_ref, s_ref, out_ref, idx_v, cidx_v, s_v, hist_v):
    w = lax.axis_index("s") * NC + lax.axis_index("c")
    pltpu.sync_copy(row_ref.at[w], idx_v)
    pltpu.sync_copy(col_ref.at[w], cidx_v)
    pltpu.sync_copy(s_ref, s_v)

    def val(base):
        cv = cidx_v[pl.ds(base, LANES)]
        return plsc.load_gather(s_v, [cv])

    _hist_loop(idx_v, hist_v, val)
    pltpu.sync_copy(hist_v, out_ref.at[w])


_rhist = pl.kernel(
    _rhist_body,
    out_type=jax.ShapeDtypeStruct((NW, NPAD), jnp.float32),
    mesh=_MESH,
    compiler_params=pltpu.CompilerParams(needs_layout_passes=False),
    scratch_types=[
        pltpu.VMEM((EPW,), jnp.int32),
        pltpu.VMEM((EPW,), jnp.int32),
        pltpu.VMEM((NPAD,), jnp.float32),
        pltpu.VMEM((NPAD,), jnp.float32),
    ],
)


def _spmm_body(g_ref, col_ref, row_ref, out_ref,
               cbuf, rbuf, dbuf, obuf, zbuf, acc, *gsems):
    c = lax.axis_index("c")
    s = lax.axis_index("s")
    zero16 = jnp.zeros((LANES,), jnp.float32)
    nz = FBW // LANES

    def zb(i, _):
        zbuf[i // nz, pl.ds((i % nz) * LANES, LANES)] = zero16
        return ()

    lax.fori_loop(0, RCHUNK * nz, zb, ())
    pltpu.sync_copy(row_ref.at[s], rbuf)

    for j in range(2):
        pltpu.sync_copy(col_ref.at[j, c, s], cbuf)
        for k in range(RPT // RCHUNK):
            pltpu.sync_copy(zbuf, acc.at[pl.ds(s * RPT + k * RCHUNK, RCHUNK)])
        plsc.subcore_barrier()

        for b in range(NBUF):
            pltpu.async_copy(g_ref.at[cbuf.at[b]], dbuf.at[b], gsems[b])

        def grp(gi, _):
            for b in range(NBUF):
                i = gi * NBUF + b
                pltpu.make_async_copy(
                    g_ref.at[cbuf.at[i]], dbuf.at[b], gsems[b]
                ).wait()
                pltpu.sync_copy(dbuf.at[b], acc.at[rbuf.at[i]], add=True)

                @pl.when(i + NBUF < NCHUNK)
                def _():
                    pltpu.async_copy(
                        g_ref.at[cbuf.at[i + NBUF]], dbuf.at[b], gsems[b]
                    )
            return ()

        lax.fori_loop(0, NCHUNK // NBUF, grp, ())
        plsc.subcore_barrier()
        for k in range(RPT // RCHUNK):
            pltpu.sync_copy(acc.at[pl.ds(s * RPT + k * RCHUNK, RCHUNK)], obuf)
            pltpu.sync_copy(
                obuf, out_ref.at[j, c, pl.ds(s * RPT + k * RCHUNK, RCHUNK)]
            )


_spmm = pl.kernel(
    _spmm_body,
    out_type=jax.ShapeDtypeStruct((2, NC, NPAD, FBW), jnp.float32),
    mesh=_MESH,
    compiler_params=pltpu.CompilerParams(
        needs_layout_passes=False, use_tc_tiling_on_sc=False
    ),
    scratch_types=[
        pltpu.VMEM((NCHUNK, CH), jnp.int32),
        pltpu.VMEM((NCHUNK, CH), jnp.int32),
        pltpu.VMEM((NBUF, CH, FBW), jnp.float32),
        pltpu.VMEM((RCHUNK, FBW), jnp.float32),
        pltpu.VMEM((RCHUNK, FBW), jnp.float32),
        pltpu.VMEM_SHARED((NPAD, FBW), jnp.float32),
    ] + [pltpu.SemaphoreType.DMA] * NBUF,
)


def _l0_body(pt_ref, x_ref, g_ref, s_ref, sv_ref):
    deg = jnp.sum(pt_ref[...], axis=1, keepdims=True) + 1.0
    scol = lax.rsqrt(deg)
    g_ref[0] = scol * x_ref[...]
    s_ref[...] = jnp.broadcast_to(scol, (RT, 128))
    sv_ref[...] = scol


def _mid0_body(t_ref, g_ref, s_ref, rp_ref, w_ref, b_ref, o_ref):
    sb = s_ref[...]
    s1 = sb[:, 0:1]
    r = s1 * (jnp.sum(rp_ref[...], axis=1, keepdims=True) + s1)
    acc = jnp.zeros((RT, 128), jnp.float32)
    for fb in range(2):
        t = jnp.concatenate([t_ref[fb, 0], t_ref[fb, 1]], axis=1)
        u = sb * (t + g_ref[fb])
        acc = acc + jnp.dot(
            u, w_ref[...][fb * 128:(fb + 1) * 128, :],
            preferred_element_type=jnp.float32,
        )
    o_ref[0] = jnp.maximum(acc + r * b_ref[...], 0.0)


def _mm_body(z_ref, s_ref, w_ref, b_ref, o_ref):
    acc = jnp.zeros((RT, 128), jnp.float32)
    for fb in range(4):
        acc = acc + jnp.dot(
            z_ref[fb], w_ref[...][fb * 128:(fb + 1) * 128, :],
            preferred_element_type=jnp.float32,
        )
    o_ref[0] = s_ref[...] * (acc + b_ref[...])


def _mid_body(ta_ref, tb_ref, g_ref, s_ref, w_ref, b_ref, o_ref):
    sb = s_ref[...]
    acc = jnp.zeros((RT, 128), jnp.float32)
    for fb in range(4):
        tr = ta_ref if fb < 2 else tb_ref
        t = jnp.concatenate([tr[fb % 2, 0], tr[fb % 2, 1]], axis=1)
        z = jnp.maximum(sb * (t + g_ref[fb]), 0.0)
        acc = acc + jnp.dot(
            z, w_ref[...][fb * 128:(fb + 1) * 128, :],
            preferred_element_type=jnp.float32,
        )
    o_ref[0] = sb * (acc + b_ref[...])


def _fin_body(t_ref, g_ref, s_ref, o_ref):
    for k in range(2):
        t = jnp.concatenate([t_ref[k, 0], t_ref[k, 1]], axis=1)
        o_ref[:, pl.ds(k * 128, 128)] = s_ref[...] * (t + g_ref[k])


_NT = NPAD // RT
_T2SPEC = pl.BlockSpec((2, NC, RT, FBW), lambda k, i: (0, 0, i, 0))

_l0 = pl.pallas_call(
    _l0_body,
    grid=(NFEAT // 128, _NT),
    in_specs=[
        pl.BlockSpec((RT, NW), lambda k, i: (i, 0)),
        pl.BlockSpec((RT, 128), lambda k, i: (i, k)),
    ],
    out_specs=[
        pl.BlockSpec((1, RT, 128), lambda k, i: (k, i, 0)),
        pl.BlockSpec((RT, 128), lambda k, i: (i, 0)),
        pl.BlockSpec((RT, 1), lambda k, i: (i, 0)),
    ],
    out_shape=[
        jax.ShapeDtypeStruct((4, NPAD, 128), jnp.float32),
        jax.ShapeDtypeStruct((NPAD, 128), jnp.float32),
        jax.ShapeDtypeStruct((NPAD, 1), jnp.float32),
    ],
)

_mid0 = pl.pallas_call(
    _mid0_body,
    grid=(NHID // 128, _NT),
    in_specs=[
        _T2SPEC,
        pl.BlockSpec((NFEAT // 128, RT, 128), lambda k, i: (0, i, 0)),
        pl.BlockSpec((RT, 128), lambda k, i: (i, 0)),
        pl.BlockSpec((RT, NW), lambda k, i: (i, 0)),
        pl.BlockSpec((NFEAT, 128), lambda k, i: (0, k)),
        pl.BlockSpec((1, 128), lambda k, i: (0, k)),
    ],
    out_specs=pl.BlockSpec((1, RT, 128), lambda k, i: (k, i, 0)),
    out_shape=jax.ShapeDtypeStruct((NHID // 128, NPAD, 128), jnp.float32),
)

_mm1 = pl.pallas_call(
    _mm_body,
    grid=(NHID // 128, _NT),
    in_specs=[
        pl.BlockSpec((NHID // 128, RT, 128), lambda k, i: (0, i, 0)),
        pl.BlockSpec((RT, 128), lambda k, i: (i, 0)),
        pl.BlockSpec((NHID, 128), lambda k, i: (0, k)),
        pl.BlockSpec((1, 128), lambda k, i: (0, k)),
    ],
    out_specs=pl.BlockSpec((1, RT, 128), lambda k, i: (k, i, 0)),
    out_shape=jax.ShapeDtypeStruct((NHID // 128, NPAD, 128), jnp.float32),
)


def _make_mid(kout):
    return pl.pallas_call(
        _mid_body,
        grid=(kout, _NT),
        in_specs=[
            _T2SPEC, _T2SPEC,
            pl.BlockSpec((NHID // 128, RT, 128), lambda k, i: (0, i, 0)),
            pl.BlockSpec((RT, 128), lambda k, i: (i, 0)),
            pl.BlockSpec((NHID, 128), lambda k, i: (0, k)),
            pl.BlockSpec((1, 128), lambda k, i: (0, k)),
        ],
        out_specs=pl.BlockSpec((1, RT, 128), lambda k, i: (k, i, 0)),
        out_shape=jax.ShapeDtypeStruct((NHID // 128, NPAD, 128), jnp.float32),
    )


_mid2 = _make_mid(NCLASS // 128)

_fin = pl.pallas_call(
    _fin_body,
    grid=(_NT,),
    in_specs=[
        pl.BlockSpec((2, NC, RT, FBW), lambda i: (0, 0, i, 0)),
        pl.BlockSpec((NCLASS // 128, RT, 128), lambda i: (0, i, 0)),
        pl.BlockSpec((RT, 128), lambda i: (i, 0)),
    ],
    out_specs=pl.BlockSpec((RT, NCLASS), lambda i: (i, 0)),
    out_shape=jax.ShapeDtypeStruct((NPAD, NCLASS), jnp.float32),
)


def kernel(x, edge_index, W0, b0, W1, b1, W2, b2):
    row = edge_index[0].astype(jnp.int32)
    col = edge_index[1].astype(jnp.int32)
    row_h = row.reshape(NW, EPW)
    col_h = col.reshape(NW, EPW)
    row_rs = row.reshape(NS, NCHUNK, CH)
    col2 = 2 * col

    def cidx(k0, k1):
        per = jnp.stack([
            jnp.stack([col2 + (2 * k * NPAD + c) for c in range(NC)])
            for k in (k0, k1)
        ])
        return per.reshape(2, NC, NS, NCHUNK, CH)

    c01, c23 = cidx(0, 1), cidx(2, 3)
    x_pad = jnp.pad(x, ((0, NPAD - N), (0, 0)))

    partials_t = _hist(row_h).T
    g0, sblk, svec = _l0(partials_t, x_pad)
    rpart_t = _rhist(row_h, col_h, svec.reshape(NPAD)).T
    t0 = _spmm(g0.reshape(8 * NPAD, FBW), c01, row_rs)
    z0 = _mid0(t0, g0, sblk, rpart_t, W0, b0.reshape(1, NHID))
    g1 = _mm1(z0, sblk, W1, b1.reshape(1, NHID))
    g1f = g1.reshape(8 * NPAD, FBW)
    t1a = _spmm(g1f, c01, row_rs)
    t1b = _spmm(g1f, c23, row_rs)
    g2 = _mid2(t1a, t1b, g1, sblk, W2, b2.reshape(1, NCLASS))
    t2 = _spmm(g2.reshape(8 * NPAD, FBW), c01, row_rs)
    out = _fin(t2, g2, sblk)
    return out[:N]
